# baseline (speedup 1.0000x reference)
import math

import jax
import jax.numpy as jnp
from jax import lax
from jax.experimental import pallas as pl
from jax.experimental.pallas import tpu as pltpu
from jax.experimental.pallas import tpu_sc as plsc

_DIM = 32
_HALF = 16
_BLK = 3584
_SB = 8


def _round_up(n, m):
    return ((n + m - 1) // m) * m


def _segsum_sorted(tables, srcs, dsts, n_pad, capc, width):
    nq = len(tables)
    half = n_pad // 2
    npt = half // 16
    acc_rows = _round_up(half + 1, 2048)
    nzero = npt // 32

    def body(*refs):
        src0, src1, dst0, dst1 = refs[:4]
        tbls = refs[4:4 + nq]
        out_hbm = refs[4 + nq]
        sbuf, dbuf, rows, zbuf, acc, sem = refs[5 + nq:]
        c = lax.axis_index("c")
        s = lax.axis_index("s")

        def _zb(i, carry):
            for k in range(width // 16):
                zbuf[i, pl.ds(16 * k, 16)] = jnp.zeros((16,), jnp.float32)
            return carry
        lax.fori_loop(0, 32, _zb, 0)

        def emit_units(src_h, dst_h, r):
            for g in range(nq):
                def _zc(i, carry):
                    pltpu.sync_copy(zbuf, acc.at[pl.ds(s * npt + i * 32, 32)])
                    return carry
                lax.fori_loop(0, nzero, _zc, 0)

                def _super(gg, carry):
                    base = s * capc + gg * _SB
                    pltpu.sync_copy(src_h.at[pl.ds(base, _SB)], sbuf)
                    pltpu.sync_copy(dst_h.at[pl.ds(base, _SB)], dbuf)

                    def _chunk(j, carry2):
                        pltpu.async_copy(tbls[g].at[sbuf.at[j]], rows,
                                         sem).wait()
                        pltpu.sync_copy(rows, acc.at[dbuf.at[j]], add=True)
                        return carry2
                    lax.fori_loop(0, _SB, _chunk, 0)
                    return carry
                lax.fori_loop(0, capc // _SB, _super, 0)

                pltpu.sync_copy(
                    acc.at[pl.ds(s * npt, npt)],
                    out_hbm.at[pl.ds(r * half + s * npt, npt),
                               pl.ds(g * width, width)])

        @pl.when(c == 0)
        def _():
            emit_units(src0, dst0, 0)

        @pl.when(c == 1)
        def _():
            emit_units(src1, dst1, 1)

    fn = pl.kernel(
        body,
        mesh=plsc.VectorSubcoreMesh(core_axis_name="c", subcore_axis_name="s"),
        compiler_params=pltpu.CompilerParams(use_tc_tiling_on_sc=False),
        out_type=jax.ShapeDtypeStruct((n_pad, nq * width), jnp.float32),
        scratch_types=[
            pltpu.VMEM((_SB, 128), jnp.int32),
            pltpu.VMEM((_SB, 128), jnp.int32),
            pltpu.VMEM((128, width), jnp.float32),
            pltpu.VMEM((32, width), jnp.float32),
            pltpu.VMEM_SHARED((acc_rows, width), jnp.float32),
            pltpu.SemaphoreType.DMA,
        ],
    )
    return fn(srcs[0], srcs[1], dsts[0], dsts[1], *tables)


def _pool_sorted(table, srcs, dsts, b, capc):
    npt = b // 32

    def body(src0, src1, dst0, dst1, table_hbm, out_hbm,
             sbuf, dbuf, rows, zbuf, acc, sem):
        c = lax.axis_index("c")
        s = lax.axis_index("s")

        def _zb(i, carry):
            zbuf[i, pl.ds(0, 16)] = jnp.zeros((16,), jnp.float32)
            zbuf[i, pl.ds(16, 16)] = jnp.zeros((16,), jnp.float32)
            return carry
        lax.fori_loop(0, 32, _zb, 0)

        def emit(src_h, dst_h, r):
            pltpu.sync_copy(zbuf, acc.at[pl.ds(s * npt, 32)])

            def _super(gg, carry):
                base = s * capc + gg * _SB
                pltpu.sync_copy(src_h.at[pl.ds(base, _SB)], sbuf)
                pltpu.sync_copy(dst_h.at[pl.ds(base, _SB)], dbuf)

                def _chunk(j, carry2):
                    pltpu.async_copy(table_hbm.at[sbuf.at[j]], rows,
                                     sem).wait()
                    pltpu.sync_copy(rows, acc.at[dbuf.at[j]], add=True)
                    return carry2
                lax.fori_loop(0, _SB, _chunk, 0)
                return carry
            lax.fori_loop(0, capc // _SB, _super, 0)

            pltpu.sync_copy(acc.at[pl.ds(s * npt, npt)],
                            out_hbm.at[pl.ds(r * (b // 2) + s * npt, npt)])

        @pl.when(c == 0)
        def _():
            emit(src0, dst0, 0)

        @pl.when(c == 1)
        def _():
            emit(src1, dst1, 1)

    fn = pl.kernel(
        body,
        mesh=plsc.VectorSubcoreMesh(core_axis_name="c", subcore_axis_name="s"),
        compiler_params=pltpu.CompilerParams(use_tc_tiling_on_sc=False),
        out_type=jax.ShapeDtypeStruct((b, _DIM), jnp.float32),
        scratch_types=[
            pltpu.VMEM((_SB, 128), jnp.int32),
            pltpu.VMEM((_SB, 128), jnp.int32),
            pltpu.VMEM((128, _DIM), jnp.float32),
            pltpu.VMEM((32, _DIM), jnp.float32),
            pltpu.VMEM_SHARED((2048, _DIM), jnp.float32),
            pltpu.SemaphoreType.DMA,
        ],
    )
    return fn(srcs[0], srcs[1], dsts[0], dsts[1], table)


def _bdot(a, b):
    return jnp.dot(a.astype(jnp.bfloat16), b.astype(jnp.bfloat16),
                   preferred_element_type=jnp.float32)


def _layer_tail(h, p_ref, w1_ref, w2_ref):
    u = jnp.maximum(_bdot(h, w1_ref[...]) + p_ref[0:1, :], 0.0)
    v = jnp.maximum(_bdot(u, w2_ref[...]) + p_ref[1:2, :], 0.0)
    return v * p_ref[2:3, :] + p_ref[3:4, :]


def _emit_layer(body, n, in_specs, args, last):
    if last:
        out_specs = pl.BlockSpec((_BLK, _DIM), lambda i: (i, 0))
        out_shape = jax.ShapeDtypeStruct((n, _DIM), jnp.float32)
    else:
        out_specs = [pl.BlockSpec((_BLK, _HALF), lambda i: (i, 0)),
                     pl.BlockSpec((_BLK, _HALF), lambda i: (i, 0))]
        out_shape = [jax.ShapeDtypeStruct((n, _HALF), jnp.float32),
                     jax.ShapeDtypeStruct((n, _HALF), jnp.float32)]
    return pl.pallas_call(
        body, grid=(n // _BLK,), in_specs=in_specs,
        out_specs=out_specs, out_shape=out_shape)(*args)


def _store_split(x1, rest, last):
    if last:
        (o_ref,) = rest
        o_ref[...] = x1
    else:
        lo_ref, hi_ref = rest
        lo_ref[...] = x1[:, :_HALF]
        hi_ref[...] = x1[:, _HALF:]


def _tc_layer1(xa, xb, agg, pvec, w1p, w2, last=False):
    n = xa.shape[0]
    aw = agg.shape[1]

    def body(xa_ref, xb_ref, a_ref, p_ref, w1_ref, w2_ref, *rest):
        a = a_ref[...]
        if aw < 64:
            a = jnp.concatenate(
                [a, jnp.zeros((a.shape[0], 64 - aw), jnp.float32)], axis=1)
        h = jnp.concatenate([xa_ref[...], xb_ref[...]], axis=1) + a
        _store_split(_layer_tail(h, p_ref, w1_ref, w2_ref), rest, last)

    in_specs = [pl.BlockSpec((_BLK, _DIM), lambda i: (i, 0)),
                pl.BlockSpec((_BLK, _DIM), lambda i: (i, 0)),
                pl.BlockSpec((_BLK, aw), lambda i: (i, 0)),
                pl.BlockSpec((8, _DIM), lambda i: (0, 0)),
                pl.BlockSpec((2 * _DIM, _DIM), lambda i: (0, 0)),
                pl.BlockSpec((_DIM, _DIM), lambda i: (0, 0))]
    return _emit_layer(body, n, in_specs, [xa, xb, agg, pvec, w1p, w2], last)


def _tc_layer(x_lo, x_hi, agg, pvec, w1, w2, last=False):
    n = agg.shape[0]

    def body(xl_ref, xh_ref, a_ref, p_ref, w1_ref, w2_ref, *rest):
        h = jnp.concatenate([xl_ref[...], xh_ref[...]], axis=1) + a_ref[...]
        _store_split(_layer_tail(h, p_ref, w1_ref, w2_ref), rest, last)

    in_specs = [pl.BlockSpec((_BLK, _HALF), lambda i: (i, 0)),
                pl.BlockSpec((_BLK, _HALF), lambda i: (i, 0)),
                pl.BlockSpec((_BLK, _DIM), lambda i: (i, 0)),
                pl.BlockSpec((8, _DIM), lambda i: (0, 0)),
                pl.BlockSpec((_DIM, _DIM), lambda i: (0, 0)),
                pl.BlockSpec((_DIM, _DIM), lambda i: (0, 0))]
    return _emit_layer(body, n, in_specs, [x_lo, x_hi, agg, pvec, w1, w2],
                       last)


def _tc_head(pd, pt, wfd, bfd, wft, bft, w1, b1, w2, b2, w3p, b3p):
    def body(pd_ref, pt_ref, wfd_ref, bfd_ref, wft_ref, bft_ref,
             w1_ref, b1_ref, w2_ref, b2_ref, w3_ref, b3_ref, o_ref):
        hd = jnp.maximum(_bdot(pd_ref[...], wfd_ref[...]) + bfd_ref[...], 0.0)
        ht = jnp.maximum(_bdot(pt_ref[...], wft_ref[...]) + bft_ref[...], 0.0)
        xj = jnp.concatenate([hd, ht], axis=1)
        h = jnp.maximum(_bdot(xj, w1_ref[...]) + b1_ref[...], 0.0)
        h = jnp.maximum(_bdot(h, w2_ref[...]) + b2_ref[...], 0.0)
        o_ref[...] = _bdot(h, w3_ref[...]) + b3_ref[...]

    return pl.pallas_call(
        body,
        out_shape=jax.ShapeDtypeStruct((pd.shape[0], 128), jnp.float32),
    )(pd, pt, wfd, bfd, wft, bft, w1, b1, w2, b2, w3p, b3p)


def _tile_ranges(sorted_keys, cut_step, cap):
    cuts = jnp.searchsorted(
        sorted_keys,
        jnp.arange(33, dtype=jnp.int32) * cut_step).astype(jnp.int32)
    starts, ends = cuts[:32], cuts[1:]
    pos = starts[:, None] + jnp.arange(cap, dtype=jnp.int32)[None, :]
    valid = pos < ends[:, None]
    return pos, valid


_RMAX = 64


def _edge_setup(ei, n_pad, cap):
    src, dst = ei[0], ei[1]
    e = src.shape[0]
    order = jnp.argsort(dst, stable=True)
    src_s, dst_s = src[order], dst[order]
    half, npt = n_pad // 2, n_pad // 32
    tile = dst_s // npt
    rank = jnp.arange(e, dtype=jnp.int32) - jnp.searchsorted(
        dst_s, dst_s).astype(jnp.int32)
    key = tile * _RMAX + jnp.minimum(rank, _RMAX - 1)
    order2 = jnp.argsort(key, stable=True)
    src2, dst2 = src_s[order2], dst_s[order2]
    g = jnp.bincount(key, length=32 * _RMAX).astype(jnp.int32)
    gstart = (jnp.cumsum(g) - g).reshape(32, _RMAX)
    g2 = g.reshape(32, _RMAX)
    padded = ((g2 + 127) // 128) * 128
    off = jnp.cumsum(padded, axis=1) - padded
    slot = jnp.arange(cap, dtype=jnp.int32)
    r_of = jax.vmap(
        lambda o: jnp.searchsorted(o, slot, side="right"))(off).astype(
            jnp.int32) - 1
    off_r = jnp.take_along_axis(off, r_of, axis=1)
    pos_in = slot[None, :] - off_r
    valid = pos_in < jnp.take_along_axis(g2, r_of, axis=1)
    gidx = jnp.minimum(
        jnp.take_along_axis(gstart, r_of, axis=1) + pos_in, e - 1)
    rowbase = (jnp.arange(32, dtype=jnp.int32)[:, None] // 16) * half
    src_t = jnp.where(valid, jnp.take(src2, gidx), 0)
    dst_t = jnp.where(valid, jnp.take(dst2, gidx) - rowbase, half)
    srcs = [src_t[r * 16:(r + 1) * 16].reshape(-1, 128) for r in (0, 1)]
    dsts = [dst_t[r * 16:(r + 1) * 16].reshape(-1, 128) for r in (0, 1)]
    return srcs, dsts


def _pool_setup(batch, n, b, cap):
    batch = batch.astype(jnp.int32)
    pos, valid = _tile_ranges(batch, b // 32, cap)
    posc = jnp.minimum(pos, n - 1)
    rowbase = (jnp.arange(32, dtype=jnp.int32)[:, None] // 16) * (b // 2)
    src_t = jnp.where(valid, posc, 0)
    dst_t = jnp.where(valid, jnp.take(batch, posc) - rowbase, b // 2)
    srcs = [src_t[r * 16:(r + 1) * 16].reshape(-1, 128) for r in (0, 1)]
    dsts = [dst_t[r * 16:(r + 1) * 16].reshape(-1, 128) for r in (0, 1)]
    return srcs, dsts


def _branch(x, ei, batch, params, name, b):
    n, f = x.shape
    e = ei.shape[1]
    n_pad = _round_up(n, _BLK)
    nq = (f + _HALF - 1) // _HALF
    x64 = jnp.pad(x, ((0, n_pad - n), (0, 64 - f)))

    cap = _round_up(e // 32 + e // 288 + _RMAX * 128 + 2048, 1024)
    srcs, dsts = _edge_setup(ei, n_pad, cap)
    capc = cap // 128

    q = [x64[:, g * _HALF:(g + 1) * _HALF] for g in range(nq)]
    agg1 = _segsum_sorted(q, srcs, dsts, n_pad, capc, _HALF)

    def pv(l):
        cv = params[name + "_conv" + str(l)]
        bn = params[name + "_bn" + str(l)]
        p = jnp.zeros((8, _DIM), jnp.float32)
        p = p.at[0].set(cv["b1"]).at[1].set(cv["b2"])
        p = p.at[2].set(bn["gamma"] / jnp.sqrt(jnp.float32(1.0 + 1e-5)))
        p = p.at[3].set(bn["beta"])
        return p, cv["W1"], cv["W2"]

    p1, w1, w2 = pv(1)
    w1p = jnp.pad(w1, ((0, 64 - f), (0, 0)))
    x_lo, x_hi = _tc_layer1(x64[:, :_DIM], x64[:, _DIM:], agg1, p1, w1p, w2)
    for l in range(2, 6):
        agg = _segsum_sorted([x_lo, x_hi], srcs, dsts, n_pad, capc, _HALF)
        pl_, w1, w2 = pv(l)
        out = _tc_layer(x_lo, x_hi, agg, pl_, w1, w2, last=(l == 5))
        if l < 5:
            x_lo, x_hi = out
        else:
            x5 = out

    capp = _round_up(n // 32 + max(1024, n // 288), 1024)
    psrcs, pdsts = _pool_setup(batch, n, b, capp)
    return _pool_sorted(x5, psrcs, pdsts, b, capp // 128)


def kernel(xd, xd_edge_index, xd_batch, xt, xt_edge_index, xt_batch, y, params):
    b = y.shape[0]
    pool_d = _branch(xd, xd_edge_index, xd_batch, params, "xd", b)
    pool_t = _branch(xt, xt_edge_index, xt_batch, params, "xt", b)

    fcd, fct, c = params["xd_fc"], params["xt_fc"], params["cls"]
    w3p = jnp.pad(c["W3"], ((0, 0), (0, 127)))
    b3p = jnp.pad(c["b3"], (0, 127)).reshape(1, 128)
    out = _tc_head(pool_d, pool_t,
                   fcd["W"], fcd["b"].reshape(1, 128),
                   fct["W"], fct["b"].reshape(1, 128),
                   c["W1"], c["b1"].reshape(1, 1024),
                   c["W2"], c["b2"].reshape(1, 256),
                   w3p, b3p)
    return (out[:, 0], y)
